# baseline (speedup 1.0000x reference)
import jax
import jax.numpy as jnp
from jax import lax
from jax.experimental import pallas as pl
from jax.experimental.pallas import tpu as pltpu
from jax.experimental.pallas import tpu_sc as plsc

B = 16384
F = 26
FP = 32
NW = 32
RPW = B // NW
CH = 16
GPC = CH * FP // 128
NCH = RPW // CH
IRW = RPW * FP // 128


def _fm_body(sidx_hbm, idx_hbm, emb_hbm, lin_hbm, bias_hbm, out_hbm,
             sidx_v, idx_v, emb_v, lin_v, out_v, bias_v, sem):
    w = lax.axis_index("s") * 2 + lax.axis_index("c")
    lanes = lax.iota(jnp.int32, 16)
    lane0 = lanes == 0
    tail_mask = lanes < (F - 16)

    pltpu.sync_copy(sidx_hbm.at[pl.ds(w * IRW, IRW)], sidx_v)
    pltpu.sync_copy(idx_hbm.at[pl.ds(w * IRW, IRW)], idx_v)
    pltpu.sync_copy(bias_hbm, bias_v.at[pl.ds(0, 1)])
    bias_lane0 = jnp.where(lane0, bias_v[...], 0.0)

    def chunk_body(c, carry):
        copies = []
        for j in range(GPC):
            g = c * GPC + j
            copies.append(pltpu.make_async_copy(
                emb_hbm.at[sidx_v.at[g]], emb_v.at[pl.ds(j * 128, 128)], sem))
            copies.append(pltpu.make_async_copy(
                lin_hbm.at[idx_v.at[g]], lin_v.at[pl.ds(j * 128, 128)], sem))
        for cp in copies:
            cp.start()
        for cp in copies:
            cp.wait()

        def row_body(b, carry2):
            iv0 = idx_v[c * GPC + (b >> 2), pl.ds((b & 3) * FP, 16)]
            iv1 = idx_v[c * GPC + (b >> 2), pl.ds((b & 3) * FP + 16, 16)]
            ov0 = (iv0 & 3) << 5
            ov1 = (iv1 & 3) << 5
            s0 = jnp.zeros((16,), jnp.float32)
            s1 = jnp.zeros((16,), jnp.float32)
            q0 = jnp.zeros((16,), jnp.float32)
            q1 = jnp.zeros((16,), jnp.float32)
            for f in range(F):
                o = ov0[f] if f < 16 else ov1[f - 16]
                r = b * FP + f
                v0 = emb_v[r, pl.ds(o, 16)]
                v1 = emb_v[r, pl.ds(o + 16, 16)]
                s0 = s0 + v0
                s1 = s1 + v1
                q0 = q0 + v0 * v0
                q1 = q1 + v1 * v1
            lv0 = lin_v[pl.ds(b * FP, 16)]
            lv1 = jnp.where(tail_mask, lin_v[pl.ds(b * FP + 16, 16)], 0.0)
            t = (s0 * s0 - q0 + s1 * s1 - q1) * 0.5
            total = jnp.sum(t + lv0 + lv1 + bias_lane0)
            plsc.store_scatter(out_v, [jnp.broadcast_to(c * CH + b, (16,))],
                               jnp.broadcast_to(total, (16,)), mask=lane0)
            return carry2

        lax.fori_loop(0, CH, row_body, 0)
        return carry

    lax.fori_loop(0, NCH, chunk_body, 0)
    pltpu.sync_copy(out_v, out_hbm.at[pl.ds(w * RPW, RPW)])


VOC = 1000000


CB = 512
NFULL = VOC // CB
KPW = NFULL // NW
XBLK = NW * KPW
TAIL = VOC - NFULL * CB
TBASE = NFULL * CB
GRP = 32 * CB // 16
UNR = 8


def _transpose_body(ewt_hbm, out_hbm, blk_v0, blk_v1, ob_v0, ob_v1,
                    si0, si1, so0, so1):
    w = lax.axis_index("s") * 2 + lax.axis_index("c")
    lanes = lax.iota(jnp.int32, 16)
    cid_base = (lanes & 3) << 5
    rid_base = lanes >> 2

    bufs = [(blk_v0, ob_v0, si0, so0), (blk_v1, ob_v1, si1, so1)]

    def in_copy(k, p):
        blk = w + NW * k
        return pltpu.make_async_copy(
            ewt_hbm.at[pl.ds(0, 32), pl.ds(blk * CB, CB)], bufs[p][0],
            bufs[p][2])

    def out_copy(k, p):
        blk = w + NW * k
        return pltpu.make_async_copy(
            bufs[p][1], out_hbm.at[pl.ds(blk * (CB // 4), CB // 4)],
            bufs[p][3])

    def transpose_block(bv, ov):
        def grp_body(i, carry):
            d = i >> 2
            c0 = (i & 3) << 7
            cid = cid_base + d
            rid0 = rid_base + (c0 >> 2)
            for u in range(UNR):
                v = bv[d, pl.ds(c0 + 16 * u, 16)]
                plsc.store_scatter(ov, [rid0 + 4 * u, cid], v)
            return carry

        lax.fori_loop(0, GRP // UNR, grp_body, 0)

    in_copy(0, 0).start()
    in_copy(1, 1).start()

    def pair_body(j, carry):
        b0 = 2 * j
        in_copy(b0, 0).wait()

        @pl.when(j >= 1)
        def _():
            out_copy(b0 - 2, 0).wait()

        transpose_block(blk_v0, ob_v0)
        out_copy(b0, 0).start()
        in_copy(b0 + 2, 0).start()

        in_copy(b0 + 1, 1).wait()

        @pl.when(j >= 1)
        def _():
            out_copy(b0 - 1, 1).wait()

        transpose_block(blk_v1, ob_v1)
        out_copy(b0 + 1, 1).start()

        @pl.when(j < (KPW - 1) // 2 - 1)
        def _():
            in_copy(b0 + 3, 1).start()

        return carry

    lax.fori_loop(0, (KPW - 1) // 2, pair_body, 0)
    in_copy(KPW - 1, 0).wait()
    out_copy(KPW - 3, 0).wait()
    transpose_block(blk_v0, ob_v0)
    out_copy(KPW - 1, 0).start()
    out_copy(KPW - 2, 1).wait()
    out_copy(KPW - 1, 0).wait()

    @pl.when(w == 0)
    def _():
        pltpu.sync_copy(ewt_hbm.at[pl.ds(0, 32), pl.ds(XBLK * CB, CB)],
                        blk_v0)
        transpose_block(blk_v0, ob_v0)
        pltpu.sync_copy(ob_v0, out_hbm.at[pl.ds(XBLK * (CB // 4), CB // 4)])
        for d in range(32):
            pltpu.sync_copy(ewt_hbm.at[d, pl.ds(TBASE, TAIL)],
                            blk_v0.at[d, pl.ds(0, TAIL)])

        def tgrp_body(i, carry):
            d = i >> 2
            c0 = (i & 3) << 4
            v = blk_v0[d, pl.ds(c0, 16)]
            plsc.store_scatter(ob_v0, [(c0 >> 2) + rid_base, cid_base + d], v)
            return carry

        lax.fori_loop(0, 32 * TAIL // 16, tgrp_body, 0)
        pltpu.sync_copy(ob_v0.at[pl.ds(0, TAIL // 4)],
                        out_hbm.at[pl.ds(TBASE // 4, TAIL // 4)])


def _transpose_table(ewt):
    run = pl.kernel(
        _transpose_body,
        out_type=jax.ShapeDtypeStruct((VOC // 4, 128), jnp.float32),
        mesh=plsc.VectorSubcoreMesh(core_axis_name="c", subcore_axis_name="s"),
        compiler_params=pltpu.CompilerParams(needs_layout_passes=False),
        scratch_types=[
            pltpu.VMEM((32, CB), jnp.float32),
            pltpu.VMEM((32, CB), jnp.float32),
            pltpu.VMEM((CB // 4, 128), jnp.float32),
            pltpu.VMEM((CB // 4, 128), jnp.float32),
            pltpu.SemaphoreType.DMA,
            pltpu.SemaphoreType.DMA,
            pltpu.SemaphoreType.DMA,
            pltpu.SemaphoreType.DMA,
        ],
    )
    return run(ewt)


def kernel(interaction_pairs, embedding_weight, linear_weight, bias):
    idxp = jnp.concatenate(
        [interaction_pairs, interaction_pairs[:, :FP - F]], axis=1)
    idx32 = idxp.reshape(-1, 128)
    sidx = (idxp >> 2).reshape(-1, 128)
    emb4 = _transpose_table(embedding_weight.T)
    lin1 = jnp.sum(linear_weight, axis=1)
    run = pl.kernel(
        _fm_body,
        out_type=jax.ShapeDtypeStruct((B,), jnp.float32),
        mesh=plsc.VectorSubcoreMesh(core_axis_name="c", subcore_axis_name="s"),
        compiler_params=pltpu.CompilerParams(needs_layout_passes=False),
        scratch_types=[
            pltpu.VMEM((IRW, 128), jnp.int32),
            pltpu.VMEM((IRW, 128), jnp.int32),
            pltpu.VMEM((CH * FP, 128), jnp.float32),
            pltpu.VMEM((CH * FP,), jnp.float32),
            pltpu.VMEM((RPW,), jnp.float32),
            pltpu.VMEM((16,), jnp.float32),
            pltpu.SemaphoreType.DMA,
        ],
    )
    return run(sidx, idx32, emb4, lin1, bias)


# speedup vs baseline: 1.1751x; 1.1751x over previous
import jax
import jax.numpy as jnp
from jax import lax
from jax.experimental import pallas as pl
from jax.experimental.pallas import tpu as pltpu
from jax.experimental.pallas import tpu_sc as plsc

B = 16384
F = 26
FP = 32
NW = 32
RPW = B // NW
CH = 16
GPC = CH * FP // 128
NCH = RPW // CH
IRW = RPW * FP // 128


def _fm_body(sidx_hbm, idx_hbm, emb_hbm, lin_hbm, bias_hbm, out_hbm,
             sidx_v, idx_v, emb_v, lin_v, out_v, bias_v, sem):
    w = lax.axis_index("s") * 2 + lax.axis_index("c")
    lanes = lax.iota(jnp.int32, 16)
    lane0 = lanes == 0
    tail_mask = lanes < (F - 16)

    pltpu.sync_copy(sidx_hbm.at[pl.ds(w * IRW, IRW)], sidx_v)
    pltpu.sync_copy(idx_hbm.at[pl.ds(w * IRW, IRW)], idx_v)
    pltpu.sync_copy(bias_hbm, bias_v.at[pl.ds(0, 1)])
    bias_lane0 = jnp.where(lane0, bias_v[...], 0.0)

    def chunk_body(c, carry):
        copies = []
        for j in range(GPC):
            g = c * GPC + j
            copies.append(pltpu.make_async_copy(
                emb_hbm.at[sidx_v.at[g]], emb_v.at[pl.ds(j * 128, 128)], sem))
            copies.append(pltpu.make_async_copy(
                lin_hbm.at[idx_v.at[g]], lin_v.at[pl.ds(j * 128, 128)], sem))
        for cp in copies:
            cp.start()
        for cp in copies:
            cp.wait()

        def row_body(b, carry2):
            iv0 = idx_v[c * GPC + (b >> 2), pl.ds((b & 3) * FP, 16)]
            iv1 = idx_v[c * GPC + (b >> 2), pl.ds((b & 3) * FP + 16, 16)]
            ov0 = (iv0 & 3) << 5
            ov1 = (iv1 & 3) << 5
            s0 = jnp.zeros((16,), jnp.float32)
            s1 = jnp.zeros((16,), jnp.float32)
            q0 = jnp.zeros((16,), jnp.float32)
            q1 = jnp.zeros((16,), jnp.float32)
            for f in range(F):
                o = ov0[f] if f < 16 else ov1[f - 16]
                r = b * FP + f
                v0 = emb_v[r, pl.ds(o, 16)]
                v1 = emb_v[r, pl.ds(o + 16, 16)]
                s0 = s0 + v0
                s1 = s1 + v1
                q0 = q0 + v0 * v0
                q1 = q1 + v1 * v1
            lv0 = lin_v[pl.ds(b * FP, 16)]
            lv1 = jnp.where(tail_mask, lin_v[pl.ds(b * FP + 16, 16)], 0.0)
            t = (s0 * s0 - q0 + s1 * s1 - q1) * 0.5
            total = jnp.sum(t + lv0 + lv1 + bias_lane0)
            plsc.store_scatter(out_v, [jnp.broadcast_to(c * CH + b, (16,))],
                               jnp.broadcast_to(total, (16,)), mask=lane0)
            return carry2

        lax.fori_loop(0, CH, row_body, 0)
        return carry

    lax.fori_loop(0, NCH, chunk_body, 0)
    pltpu.sync_copy(out_v, out_hbm.at[pl.ds(w * RPW, RPW)])


VOC = 1000000


CB = 256
NFULL = 3904
KPW = NFULL // NW
TAIL = VOC - 3906 * CB
TBASE = 3906 * CB


def _depad_body(emb_hbm, out_hbm, blk_v0, blk_v1, ob_v0, ob_v1,
                si0, si1, so0, so1):
    w = lax.axis_index("s") * 2 + lax.axis_index("c")

    bufs = [(blk_v0, ob_v0, si0, so0), (blk_v1, ob_v1, si1, so1)]

    def in_copy(k, p):
        blk = w + NW * k
        return pltpu.make_async_copy(
            emb_hbm.at[pl.ds(blk * CB, CB)], bufs[p][0], bufs[p][2])

    def out_copy(k, p):
        blk = w + NW * k
        return pltpu.make_async_copy(
            bufs[p][1], out_hbm.at[pl.ds(blk * (CB // 4), CB // 4)],
            bufs[p][3])

    def depad_block(bv, ov, nrows):
        def row_body(i, carry):
            for u in range(4):
                r = i * 4 + u
                v0 = bv[r, pl.ds(0, 16)]
                v1 = bv[r, pl.ds(16, 16)]
                ov[r >> 2, pl.ds((r & 3) * 32, 16)] = v0
                ov[r >> 2, pl.ds((r & 3) * 32 + 16, 16)] = v1
            return carry

        lax.fori_loop(0, nrows // 4, row_body, 0)

    in_copy(0, 0).start()
    in_copy(1, 1).start()

    def pair_body(j, carry):
        b0 = 2 * j
        in_copy(b0, 0).wait()

        @pl.when(j >= 1)
        def _():
            out_copy(b0 - 2, 0).wait()

        depad_block(blk_v0, ob_v0, CB)
        out_copy(b0, 0).start()

        @pl.when(j < KPW // 2 - 1)
        def _():
            in_copy(b0 + 2, 0).start()

        in_copy(b0 + 1, 1).wait()

        @pl.when(j >= 1)
        def _():
            out_copy(b0 - 1, 1).wait()

        depad_block(blk_v1, ob_v1, CB)
        out_copy(b0 + 1, 1).start()

        @pl.when(j < KPW // 2 - 1)
        def _():
            in_copy(b0 + 3, 1).start()

        return carry

    lax.fori_loop(0, KPW // 2, pair_body, 0)
    out_copy(KPW - 2, 0).wait()
    out_copy(KPW - 1, 1).wait()

    def extra(base, nrows):
        pltpu.sync_copy(emb_hbm.at[pl.ds(base, nrows)],
                        blk_v0.at[pl.ds(0, nrows)])
        depad_block(blk_v0, ob_v0, nrows)
        pltpu.sync_copy(ob_v0.at[pl.ds(0, nrows // 4)],
                        out_hbm.at[pl.ds(base // 4, nrows // 4)])

    @pl.when(w == 0)
    def _():
        extra(3904 * CB, CB)

    @pl.when(w == 1)
    def _():
        extra(3905 * CB, CB)

    @pl.when(w == 2)
    def _():
        extra(TBASE, TAIL)


def _depad_table(emb):
    run = pl.kernel(
        _depad_body,
        out_type=jax.ShapeDtypeStruct((VOC // 4, 128), jnp.float32),
        mesh=plsc.VectorSubcoreMesh(core_axis_name="c", subcore_axis_name="s"),
        compiler_params=pltpu.CompilerParams(needs_layout_passes=False),
        scratch_types=[
            pltpu.VMEM((CB, 32), jnp.float32),
            pltpu.VMEM((CB, 32), jnp.float32),
            pltpu.VMEM((CB // 4, 128), jnp.float32),
            pltpu.VMEM((CB // 4, 128), jnp.float32),
            pltpu.SemaphoreType.DMA,
            pltpu.SemaphoreType.DMA,
            pltpu.SemaphoreType.DMA,
            pltpu.SemaphoreType.DMA,
        ],
    )
    return run(emb)


def kernel(interaction_pairs, embedding_weight, linear_weight, bias):
    idxp = jnp.concatenate(
        [interaction_pairs, interaction_pairs[:, :FP - F]], axis=1)
    idx32 = idxp.reshape(-1, 128)
    sidx = (idxp >> 2).reshape(-1, 128)
    emb4 = _depad_table(embedding_weight)
    lin1 = jnp.sum(linear_weight, axis=1)
    run = pl.kernel(
        _fm_body,
        out_type=jax.ShapeDtypeStruct((B,), jnp.float32),
        mesh=plsc.VectorSubcoreMesh(core_axis_name="c", subcore_axis_name="s"),
        compiler_params=pltpu.CompilerParams(needs_layout_passes=False),
        scratch_types=[
            pltpu.VMEM((IRW, 128), jnp.int32),
            pltpu.VMEM((IRW, 128), jnp.int32),
            pltpu.VMEM((CH * FP, 128), jnp.float32),
            pltpu.VMEM((CH * FP,), jnp.float32),
            pltpu.VMEM((RPW,), jnp.float32),
            pltpu.VMEM((16,), jnp.float32),
            pltpu.SemaphoreType.DMA,
        ],
    )
    return run(sidx, idx32, emb4, lin1, bias)


# speedup vs baseline: 1.2194x; 1.0377x over previous
import jax
import jax.numpy as jnp
from jax import lax
from jax.experimental import pallas as pl
from jax.experimental.pallas import tpu as pltpu
from jax.experimental.pallas import tpu_sc as plsc

B = 16384
F = 26
FP = 32
NW = 32
RPW = B // NW
CH = 16
GPC = CH * FP // 128
NCH = RPW // CH
IRW = RPW * FP // 128


def _fm_body(sidx_hbm, idx_hbm, emb_hbm, lin_hbm, bias_hbm, out_hbm,
             sidx_v, idx_v, emb_v, lin_v, out_v, bias_v, sem):
    w = lax.axis_index("s") * 2 + lax.axis_index("c")
    lanes = lax.iota(jnp.int32, 16)
    lane0 = lanes == 0
    tail_mask = lanes < (F - 16)

    pltpu.sync_copy(sidx_hbm.at[pl.ds(w * IRW, IRW)], sidx_v)
    pltpu.sync_copy(idx_hbm.at[pl.ds(w * IRW, IRW)], idx_v)
    pltpu.sync_copy(bias_hbm, bias_v.at[pl.ds(0, 1)])
    bias_lane0 = jnp.where(lane0, bias_v[...], 0.0)

    def chunk_body(c, carry):
        copies = []
        for j in range(GPC):
            g = c * GPC + j
            copies.append(pltpu.make_async_copy(
                emb_hbm.at[sidx_v.at[g]], emb_v.at[pl.ds(j * 128, 128)], sem))
            copies.append(pltpu.make_async_copy(
                lin_hbm.at[idx_v.at[g]], lin_v.at[pl.ds(j * 128, 128)], sem))
        for cp in copies:
            cp.start()
        for cp in copies:
            cp.wait()

        def row_body(b, carry2):
            iv0 = idx_v[c * GPC + (b >> 2), pl.ds((b & 3) * FP, 16)]
            iv1 = idx_v[c * GPC + (b >> 2), pl.ds((b & 3) * FP + 16, 16)]
            ov0 = (iv0 & 3) << 5
            ov1 = (iv1 & 3) << 5
            s0 = jnp.zeros((16,), jnp.float32)
            s1 = jnp.zeros((16,), jnp.float32)
            q0 = jnp.zeros((16,), jnp.float32)
            q1 = jnp.zeros((16,), jnp.float32)
            for f in range(F):
                o = ov0[f] if f < 16 else ov1[f - 16]
                r = b * FP + f
                v0 = emb_v[r, pl.ds(o, 16)]
                v1 = emb_v[r, pl.ds(o + 16, 16)]
                s0 = s0 + v0
                s1 = s1 + v1
                q0 = q0 + v0 * v0
                q1 = q1 + v1 * v1
            lv0 = lin_v[pl.ds(b * FP, 16)]
            lv1 = jnp.where(tail_mask, lin_v[pl.ds(b * FP + 16, 16)], 0.0)
            t = (s0 * s0 - q0 + s1 * s1 - q1) * 0.5
            total = jnp.sum(t + lv0 + lv1 + bias_lane0)
            plsc.store_scatter(out_v, [jnp.broadcast_to(c * CH + b, (16,))],
                               jnp.broadcast_to(total, (16,)), mask=lane0)
            return carry2

        lax.fori_loop(0, CH, row_body, 0)
        return carry

    lax.fori_loop(0, NCH, chunk_body, 0)
    pltpu.sync_copy(out_v, out_hbm.at[pl.ds(w * RPW, RPW)])


def kernel(interaction_pairs, embedding_weight, linear_weight, bias):
    idxp = jnp.concatenate(
        [interaction_pairs, interaction_pairs[:, :FP - F]], axis=1)
    idx32 = idxp.reshape(-1, 128)
    sidx = (idxp >> 2).reshape(-1, 128)
    emb4 = embedding_weight.reshape(-1, 128)
    lin1 = jnp.sum(linear_weight, axis=1)
    run = pl.kernel(
        _fm_body,
        out_type=jax.ShapeDtypeStruct((B,), jnp.float32),
        mesh=plsc.VectorSubcoreMesh(core_axis_name="c", subcore_axis_name="s"),
        compiler_params=pltpu.CompilerParams(needs_layout_passes=False),
        scratch_types=[
            pltpu.VMEM((IRW, 128), jnp.int32),
            pltpu.VMEM((IRW, 128), jnp.int32),
            pltpu.VMEM((CH * FP, 128), jnp.float32),
            pltpu.VMEM((CH * FP,), jnp.float32),
            pltpu.VMEM((RPW,), jnp.float32),
            pltpu.VMEM((16,), jnp.float32),
            pltpu.SemaphoreType.DMA,
        ],
    )
    return run(sidx, idx32, emb4, lin1, bias)
